# baseline (speedup 1.0000x reference)
import jax
import jax.numpy as jnp
from jax.experimental import pallas as pl
from jax.experimental.pallas import tpu as pltpu

_LOSS_WEIGHT = 1.0


def _body(p_ref, t_ref, m_ref, out_ref):
    n = pl.program_id(0)
    j = pl.program_id(1)
    nn = pl.num_programs(0)
    nj = pl.num_programs(1)

    @pl.when((n == 0) & (j == 0))
    def _init():
        out_ref[0] = 0.0
        out_ref[1] = 0.0
        out_ref[2] = 0.0

    p = p_ref[0]
    t = t_ref[0]
    m = m_ref[0]
    d = p - t
    ad = jnp.abs(d)
    sm = jnp.where(ad < 1.0, 0.5 * ad * ad, ad - 0.5)
    mb = m[None, :, :]
    out_ref[0] += jnp.sum(sm * mb)
    out_ref[1] += jnp.sum(ad * mb)
    out_ref[2] += jnp.sum(m)

    @pl.when((n == nn - 1) & (j == nj - 1))
    def _fini():
        cnt = out_ref[2] * p.shape[0]
        out_ref[0] = out_ref[0] / cnt * _LOSS_WEIGHT
        out_ref[1] = out_ref[1] / cnt


def kernel(pred, target, front_position):
    N, C, H, W = pred.shape
    HW = H * W
    assert HW % 128 == 0
    R = HW // 128
    BH = 48
    assert R % BH == 0
    NJ = R // BH

    p3 = pred.reshape(N, C, R, 128)
    t3 = target.reshape(N, C, R, 128)
    m3 = front_position.reshape(N, R, 128).astype(jnp.float32)

    out = pl.pallas_call(
        _body,
        grid=(N, NJ),
        in_specs=[
            pl.BlockSpec((1, C, BH, 128), lambda n, j: (n, 0, j, 0)),
            pl.BlockSpec((1, C, BH, 128), lambda n, j: (n, 0, j, 0)),
            pl.BlockSpec((1, BH, 128), lambda n, j: (n, j, 0)),
        ],
        out_specs=pl.BlockSpec(memory_space=pltpu.SMEM),
        out_shape=jax.ShapeDtypeStruct((3,), jnp.float32),
    )(p3, t3, m3)

    return (out[0], out[1])


# speedup vs baseline: 1.0176x; 1.0176x over previous
import jax
import jax.numpy as jnp
from jax.experimental import pallas as pl
from jax.experimental.pallas import tpu as pltpu

_LOSS_WEIGHT = 1.0


def _body(p_ref, t_ref, m_ref, out_ref, *, K, R):
    i = pl.program_id(0)
    ni = pl.num_programs(0)

    @pl.when(i == 0)
    def _init():
        out_ref[0] = 0.0
        out_ref[1] = 0.0
        out_ref[2] = 0.0

    m = m_ref[0]
    p = p_ref[...].reshape(K, R, 128)
    t = t_ref[...].reshape(K, R, 128)
    ad = jnp.abs(p - t)
    mb = m[None, :, :]
    adm = ad * mb
    cm = jnp.minimum(adm, mb)
    sm = cm * (adm - 0.5 * cm)
    out_ref[0] += jnp.sum(sm)
    out_ref[1] += jnp.sum(adm)
    out_ref[2] += jnp.sum(m)

    @pl.when(i == ni - 1)
    def _fini():
        cnt = out_ref[2] * K
        out_ref[0] = out_ref[0] / cnt * _LOSS_WEIGHT
        out_ref[1] = out_ref[1] / cnt


import functools


def kernel(pred, target, front_position):
    N, C, H, W = pred.shape
    HW = H * W
    assert HW % 128 == 0
    R = HW // 128
    K = 4
    assert C % K == 0
    steps_per_n = C // K
    nsteps = N * steps_per_n

    p2 = pred.reshape(N * C * R, 128)
    t2 = target.reshape(N * C * R, 128)
    m3 = front_position.reshape(N, R, 128).astype(jnp.float32)

    out = pl.pallas_call(
        functools.partial(_body, K=K, R=R),
        grid=(nsteps,),
        in_specs=[
            pl.BlockSpec((K * R, 128), lambda i: (i, 0)),
            pl.BlockSpec((K * R, 128), lambda i: (i, 0)),
            pl.BlockSpec((1, R, 128), lambda i: (i // steps_per_n, 0, 0)),
        ],
        out_specs=pl.BlockSpec(memory_space=pltpu.SMEM),
        out_shape=jax.ShapeDtypeStruct((3,), jnp.float32),
    )(p2, t2, m3)

    return (out[0], out[1])
